# Initial kernel scaffold; baseline (speedup 1.0000x reference)
import jax
import jax.numpy as jnp
from jax.experimental import pallas as pl


def kernel(x, params):
    raise NotImplementedError("write your pallas kernel here")


# speedup vs baseline: 6.8652x; 6.8652x over previous
import functools

import jax
import jax.numpy as jnp
from jax import lax
from jax.experimental import pallas as pl
from jax.experimental.pallas import tpu as pltpu
from jax.experimental.pallas import tpu_sc as plsc

N = 4096
K = 16
NW = 32
CH = 128
BK = 256
BL = 512
F32 = jnp.float32


def _prep_body(xt_ref, wi_ref, bi_ref, aug_ref, xf_ref):
    xt = xt_ref[...]
    c = jnp.mean(xt, axis=0, keepdims=True)
    xc = xt - c
    sq = jnp.sum(xc * xc, axis=1, keepdims=True)
    md = jnp.max(jnp.sqrt(sq))
    pos = xc / (md + 1e-8)
    psq = jnp.sum(pos * pos, axis=1, keepdims=True)
    aug_ref[...] = jnp.concatenate(
        [pos, psq, jnp.zeros((N, 15), F32)], axis=1)
    xf_ref[...] = (
        jnp.dot(pos, wi_ref[...], preferred_element_type=F32) + bi_ref[...])


def _prep(xt_pad, wi_pad, bi):
    return pl.pallas_call(
        _prep_body,
        out_shape=(
            jax.ShapeDtypeStruct((N, 32), F32),
            jax.ShapeDtypeStruct((N, 16), F32),
        ),
    )(xt_pad, wi_pad, bi)


def _knn_body(grow_ref, gt_ref, idx_ref):
    g = grow_ref[...]
    sq = g[:, 16:17]
    a = jnp.concatenate(
        [2.0 * g[:, :16], jnp.full((BK, 1), -1.0, F32),
         jnp.zeros((BK, 15), F32)], axis=1)
    pd = jnp.dot(a, gt_ref[...], preferred_element_type=F32) - sq
    colid = lax.broadcasted_iota(jnp.int32, (BK, N), 1)
    neg = jnp.float32(-jnp.inf)
    big = jnp.int32(2 ** 30)
    outs = []
    for _ in range(K):
        m = jnp.max(pd, axis=1, keepdims=True)
        it = jnp.min(jnp.where(pd == m, colid, big), axis=1, keepdims=True)
        outs.append(it)
        pd = jnp.where(colid == it, neg, pd)
    idx_ref[...] = jnp.concatenate(outs, axis=1)


def _knn(aug, aug_t):
    return pl.pallas_call(
        _knn_body,
        grid=(N // BK,),
        in_specs=[
            pl.BlockSpec((BK, 32), lambda i: (i, 0)),
            pl.BlockSpec((32, N), lambda i: (0, 0)),
        ],
        out_specs=pl.BlockSpec((BK, K), lambda i: (i, 0)),
        out_shape=jax.ShapeDtypeStruct((N, K), jnp.int32),
    )(aug, aug_t)


@functools.lru_cache(maxsize=None)
def _make_gather(d):
    rows = (N * K) // NW
    nch = rows // CH
    mesh = plsc.VectorSubcoreMesh(core_axis_name="c", subcore_axis_name="s")

    @functools.partial(
        pl.kernel, mesh=mesh,
        out_type=jax.ShapeDtypeStruct((N * K, d), F32),
        compiler_params=pltpu.CompilerParams(use_tc_tiling_on_sc=False),
        scratch_types=[
            pltpu.VMEM((CH,), jnp.int32),
            pltpu.VMEM((CH, d), F32),
            pltpu.SemaphoreType.DMA,
        ],
    )
    def gk(table_hbm, idx_hbm, out_hbm, idx_v, rows_v, sem):
        wid = lax.axis_index("s") * 2 + lax.axis_index("c")
        base = wid * rows

        def body(i, carry):
            off = base + i * CH
            pltpu.sync_copy(idx_hbm.at[pl.ds(off, CH)], idx_v)
            pltpu.async_copy(table_hbm.at[idx_v], rows_v, sem).wait()
            pltpu.sync_copy(rows_v, out_hbm.at[pl.ds(off, CH)])
            return carry

        lax.fori_loop(0, nch, body, 0)

    return gk


def _layer_body(c, x_ref, xn_ref, pn_ref, pos_ref,
                wc_ref, bc_ref, wn_ref, bn_ref, wp1_ref, bp1_ref,
                wp2_ref, bp2_ref, wa1_ref, ba1_ref, wa2_ref, ba2_ref,
                ws_ref, bs_ref, out_ref):
    b = x_ref.shape[0]
    x = x_ref[...]
    pos = pos_ref[...]
    wc = wc_ref[...]
    wn = wn_ref[...]
    wp1 = wp1_ref[...]
    wp2 = wp2_ref[...]
    wa1 = wa1_ref[...]
    wa2 = wa2_ref[...]
    ws = ws_ref[...]
    bc = bc_ref[...]
    bn = bn_ref[...]
    bp1 = bp1_ref[...]
    bp2 = bp2_ref[...]
    ba1 = ba1_ref[...]
    ba2 = ba2_ref[...]
    bs = bs_ref[...]

    fc = jnp.dot(x, wc, preferred_element_type=F32) + bc
    m = jnp.full((b, 1), -jnp.inf, F32)
    s = jnp.zeros((b, 1), F32)
    agg = jnp.zeros((b, c), F32)
    for j in range(K):
        xnj = xn_ref[j]
        pnj = pn_ref[j]
        pdj = pos - pnj
        pe = jnp.maximum(
            jnp.dot(pdj, wp1, preferred_element_type=F32) + bp1, 0.0)
        pe = jnp.dot(pe, wp2, preferred_element_type=F32) + bp2
        fn = jnp.dot(xnj, wn, preferred_element_type=F32) + bn
        t = fc - fn + pe
        h = jnp.maximum(
            jnp.dot(t, wa1, preferred_element_type=F32) + ba1, 0.0)
        h = jnp.dot(h, wa2, preferred_element_type=F32) + ba2
        lg = jnp.dot(h, ws, preferred_element_type=F32) + bs
        v = fn + pe
        nm = jnp.maximum(m, lg)
        sc = jnp.exp(m - nm)
        p = jnp.exp(lg - nm)
        s = s * sc + p
        agg = agg * sc + p * v
        m = nm
    out_ref[...] = fc + agg / s


def _layer(cin, c, feats, xn3, pn3, posp, p):
    wc, bc = p['c']
    wn, bn = p['n']
    wp1, bp1 = p['pos1']
    wp2, bp2 = p['pos2']
    wa1, ba1 = p['a1']
    wa2, ba2 = p['a2']
    ws, bs = p['score']
    wp1p = jnp.zeros((16, c), F32).at[:3].set(wp1)
    args = [feats, xn3, pn3, posp,
            wc, bc.reshape(1, c), wn, bn.reshape(1, c),
            wp1p, bp1.reshape(1, c), wp2, bp2.reshape(1, c),
            wa1, ba1.reshape(1, c), wa2, ba2.reshape(1, c),
            ws, bs.reshape(1, 1)]
    w_specs = [pl.BlockSpec(a.shape, lambda i: (0, 0)) for a in args[4:]]
    return pl.pallas_call(
        functools.partial(_layer_body, c),
        grid=(N // BL,),
        in_specs=[
            pl.BlockSpec((BL, cin), lambda i: (i, 0)),
            pl.BlockSpec((K, BL, cin), lambda i: (0, i, 0)),
            pl.BlockSpec((K, BL, 16), lambda i: (0, i, 0)),
            pl.BlockSpec((BL, 16), lambda i: (i, 0)),
        ] + w_specs,
        out_specs=pl.BlockSpec((BL, c), lambda i: (i, 0)),
        out_shape=jax.ShapeDtypeStruct((N, c), F32),
    )(*args)


def _pool_body(o1_ref, o2_ref, o3_ref, o4_ref, wfc_ref, bfc_ref, out_ref):
    cf = jnp.concatenate(
        [o1_ref[...], o2_ref[...], o3_ref[...], o4_ref[...]], axis=1)
    mx = jnp.max(cf, axis=0, keepdims=True)
    mn = jnp.mean(cf, axis=0, keepdims=True)
    g = jnp.concatenate([mx, mn], axis=1)
    out_ref[...] = (
        jnp.dot(g, wfc_ref[...], preferred_element_type=F32) + bfc_ref[...])


def _pool(o1, o2, o3, o4, wfc, bfc):
    return pl.pallas_call(
        _pool_body,
        out_shape=jax.ShapeDtypeStruct((1, 256), F32),
    )(o1, o2, o3, o4, wfc, bfc)


def kernel(x, params):
    wi, bi = params['inp']
    wi_pad = jnp.zeros((16, 16), F32).at[:3].set(wi)
    bi2 = bi.reshape(1, 16)
    wfc, bfc = params['fc']
    bfc2 = bfc.reshape(1, 256)
    layer_ps = [params['l1'], params['l2'], params['l3'], params['l4']]
    cins = [16, 16, 32, 64]
    couts = [16, 32, 64, 128]

    outs = []
    for i in range(x.shape[0]):
        xt_pad = jnp.pad(jnp.transpose(x[i]), ((0, 0), (0, 13)))
        aug, xf = _prep(xt_pad, wi_pad, bi2)
        idx = _knn(aug, jnp.transpose(aug))
        idx_flat = jnp.transpose(idx).reshape(N * K)
        posp = aug[:, :16]

        feats = xf
        pn3 = None
        os_ = []
        for li in range(4):
            cin, c = cins[li], couts[li]
            if li == 0:
                table = jnp.concatenate([feats, posp], axis=1)
                g = _make_gather(32)(table, idx_flat)
                xn3 = g[:, :16].reshape(K, N, 16)
                pn3 = g[:, 16:].reshape(K, N, 16)
            else:
                g = _make_gather(cin)(feats, idx_flat)
                xn3 = g.reshape(K, N, cin)
            feats = _layer(cin, c, feats, xn3, pn3, posp, layer_ps[li])
            os_.append(feats)
        outs.append(_pool(os_[0], os_[1], os_[2], os_[3], wfc, bfc2))
    return jnp.concatenate(outs, axis=0)


# speedup vs baseline: 6.8971x; 1.0047x over previous
import functools

import jax
import jax.numpy as jnp
from jax import lax
from jax.experimental import pallas as pl
from jax.experimental.pallas import tpu as pltpu
from jax.experimental.pallas import tpu_sc as plsc

N = 4096
K = 16
NW = 32
CH = 128
NFLY = 8
BK = 256
BL = 512
F32 = jnp.float32


def _prep_body(xt_ref, wi_ref, bi_ref, aug_ref, xf_ref):
    xt = xt_ref[0]
    c = jnp.mean(xt, axis=0, keepdims=True)
    xc = xt - c
    sq = jnp.sum(xc * xc, axis=1, keepdims=True)
    md = jnp.max(jnp.sqrt(sq))
    pos = xc / (md + 1e-8)
    psq = jnp.sum(pos * pos, axis=1, keepdims=True)
    aug_ref[0] = jnp.concatenate(
        [pos, psq, jnp.zeros((N, 15), F32)], axis=1)
    xf_ref[0] = (
        jnp.dot(pos, wi_ref[...], preferred_element_type=F32) + bi_ref[...])


def _prep(nb, xt_pad, wi_pad, bi):
    return pl.pallas_call(
        _prep_body,
        grid=(nb,),
        in_specs=[
            pl.BlockSpec((1, N, 16), lambda b: (b, 0, 0)),
            pl.BlockSpec((16, 16), lambda b: (0, 0)),
            pl.BlockSpec((1, 16), lambda b: (0, 0)),
        ],
        out_specs=(
            pl.BlockSpec((1, N, 32), lambda b: (b, 0, 0)),
            pl.BlockSpec((1, N, 16), lambda b: (b, 0, 0)),
        ),
        out_shape=(
            jax.ShapeDtypeStruct((nb, N, 32), F32),
            jax.ShapeDtypeStruct((nb, N, 16), F32),
        ),
    )(xt_pad, wi_pad, bi)


def _knn_body(grow_ref, gt_ref, idx_ref):
    g = grow_ref[0]
    sq = g[:, 16:17]
    a = jnp.concatenate(
        [2.0 * g[:, :16], jnp.full((BK, 1), -1.0, F32),
         jnp.zeros((BK, 15), F32)], axis=1)
    pd = jnp.dot(a, gt_ref[0], preferred_element_type=F32) - sq
    colid = lax.broadcasted_iota(jnp.int32, (BK, N), 1)
    neg = jnp.float32(-jnp.inf)
    big = jnp.int32(2 ** 30)
    outs = []
    for _ in range(K):
        m = jnp.max(pd, axis=1, keepdims=True)
        it = jnp.min(jnp.where(pd == m, colid, big), axis=1, keepdims=True)
        outs.append(it)
        pd = jnp.where(colid == it, neg, pd)
    idx_ref[0] = jnp.concatenate(outs, axis=1) + pl.program_id(0) * N


def _knn(nb, aug, aug_t):
    return pl.pallas_call(
        _knn_body,
        grid=(nb, N // BK),
        in_specs=[
            pl.BlockSpec((1, BK, 32), lambda b, i: (b, i, 0)),
            pl.BlockSpec((1, 32, N), lambda b, i: (b, 0, 0)),
        ],
        out_specs=pl.BlockSpec((1, BK, K), lambda b, i: (b, i, 0)),
        out_shape=jax.ShapeDtypeStruct((nb, N, K), jnp.int32),
    )(aug, aug_t)


@functools.lru_cache(maxsize=None)
def _make_gather(nrows, d):
    rows = nrows // NW
    sc_rows = CH * NFLY
    nsc = rows // sc_rows
    mesh = plsc.VectorSubcoreMesh(core_axis_name="c", subcore_axis_name="s")

    @functools.partial(
        pl.kernel, mesh=mesh,
        out_type=jax.ShapeDtypeStruct((nrows, d), F32),
        compiler_params=pltpu.CompilerParams(use_tc_tiling_on_sc=False),
        scratch_types=[
            pltpu.VMEM((rows,), jnp.int32),
            pltpu.VMEM((sc_rows, d), F32),
            pltpu.SemaphoreType.DMA,
        ],
    )
    def gk(table_hbm, idx_hbm, out_hbm, idx_v, rows_v, sem):
        wid = lax.axis_index("s") * 2 + lax.axis_index("c")
        base = wid * rows
        pltpu.sync_copy(idx_hbm.at[pl.ds(base, rows)], idx_v)

        def body(i, carry):
            off = i * sc_rows
            cps = []
            for j in range(NFLY):
                cps.append(pltpu.async_copy(
                    table_hbm.at[idx_v.at[pl.ds(off + j * CH, CH)]],
                    rows_v.at[pl.ds(j * CH, CH)], sem))
            for cp in cps:
                cp.wait()
            pltpu.sync_copy(rows_v, out_hbm.at[pl.ds(base + off, sc_rows)])
            return carry

        lax.fori_loop(0, nsc, body, 0)

    return gk


def _layer_body(c, x_ref, xn_ref, pn_ref, pos_ref,
                wc_ref, bc_ref, wn_ref, bn_ref, wp1_ref, bp1_ref,
                wp2_ref, bp2_ref, wa1_ref, ba1_ref, wa2_ref, ba2_ref,
                ws_ref, bs_ref, out_ref):
    b = x_ref.shape[0]
    x = x_ref[...]
    pos = pos_ref[...]
    wc = wc_ref[...]
    wn = wn_ref[...]
    wp1 = wp1_ref[...]
    wp2 = wp2_ref[...]
    wa1 = wa1_ref[...]
    wa2 = wa2_ref[...]
    ws = ws_ref[...]
    bc = bc_ref[...]
    bn = bn_ref[...]
    bp1 = bp1_ref[...]
    bp2 = bp2_ref[...]
    ba1 = ba1_ref[...]
    ba2 = ba2_ref[...]
    bs = bs_ref[...]

    fc = jnp.dot(x, wc, preferred_element_type=F32) + bc
    m = jnp.full((b, 1), -jnp.inf, F32)
    s = jnp.zeros((b, 1), F32)
    agg = jnp.zeros((b, c), F32)
    for j in range(K):
        xnj = xn_ref[j]
        pnj = pn_ref[j]
        pdj = pos - pnj
        pe = jnp.maximum(
            jnp.dot(pdj, wp1, preferred_element_type=F32) + bp1, 0.0)
        pe = jnp.dot(pe, wp2, preferred_element_type=F32) + bp2
        fn = jnp.dot(xnj, wn, preferred_element_type=F32) + bn
        t = fc - fn + pe
        h = jnp.maximum(
            jnp.dot(t, wa1, preferred_element_type=F32) + ba1, 0.0)
        h = jnp.dot(h, wa2, preferred_element_type=F32) + ba2
        lg = jnp.dot(h, ws, preferred_element_type=F32) + bs
        v = fn + pe
        nm = jnp.maximum(m, lg)
        sc = jnp.exp(m - nm)
        p = jnp.exp(lg - nm)
        s = s * sc + p
        agg = agg * sc + p * v
        m = nm
    out_ref[...] = fc + agg / s


def _layer(nb, cin, c, feats, xn3, pn3, posp, p):
    wc, bc = p['c']
    wn, bn = p['n']
    wp1, bp1 = p['pos1']
    wp2, bp2 = p['pos2']
    wa1, ba1 = p['a1']
    wa2, ba2 = p['a2']
    ws, bs = p['score']
    wp1p = jnp.zeros((16, c), F32).at[:3].set(wp1)
    nr = nb * N
    args = [feats, xn3, pn3, posp,
            wc, bc.reshape(1, c), wn, bn.reshape(1, c),
            wp1p, bp1.reshape(1, c), wp2, bp2.reshape(1, c),
            wa1, ba1.reshape(1, c), wa2, ba2.reshape(1, c),
            ws, bs.reshape(1, 1)]
    w_specs = [pl.BlockSpec(a.shape, lambda i: (0, 0)) for a in args[4:]]
    return pl.pallas_call(
        functools.partial(_layer_body, c),
        grid=(nr // BL,),
        in_specs=[
            pl.BlockSpec((BL, cin), lambda i: (i, 0)),
            pl.BlockSpec((K, BL, cin), lambda i: (0, i, 0)),
            pl.BlockSpec((K, BL, 16), lambda i: (0, i, 0)),
            pl.BlockSpec((BL, 16), lambda i: (i, 0)),
        ] + w_specs,
        out_specs=pl.BlockSpec((BL, c), lambda i: (i, 0)),
        out_shape=jax.ShapeDtypeStruct((nr, c), F32),
    )(*args)


def _pool_body(o1_ref, o2_ref, o3_ref, o4_ref, wfc_ref, bfc_ref, out_ref):
    cf = jnp.concatenate(
        [o1_ref[...], o2_ref[...], o3_ref[...], o4_ref[...]], axis=1)
    mx = jnp.max(cf, axis=0, keepdims=True)
    mn = jnp.mean(cf, axis=0, keepdims=True)
    g = jnp.concatenate([mx, mn], axis=1)
    out_ref[0] = (
        jnp.dot(g, wfc_ref[...], preferred_element_type=F32) + bfc_ref[...])


def _pool(nb, o1, o2, o3, o4, wfc, bfc):
    return pl.pallas_call(
        _pool_body,
        grid=(nb,),
        in_specs=[
            pl.BlockSpec((N, 16), lambda b: (b, 0)),
            pl.BlockSpec((N, 32), lambda b: (b, 0)),
            pl.BlockSpec((N, 64), lambda b: (b, 0)),
            pl.BlockSpec((N, 128), lambda b: (b, 0)),
            pl.BlockSpec((480, 256), lambda b: (0, 0)),
            pl.BlockSpec((1, 256), lambda b: (0, 0)),
        ],
        out_specs=pl.BlockSpec((1, 1, 256), lambda b: (b, 0, 0)),
        out_shape=jax.ShapeDtypeStruct((nb, 1, 256), F32),
    )(o1, o2, o3, o4, wfc, bfc)


def kernel(x, params):
    nb = x.shape[0]
    wi, bi = params['inp']
    wi_pad = jnp.zeros((16, 16), F32).at[:3].set(wi)
    bi2 = bi.reshape(1, 16)
    wfc, bfc = params['fc']
    bfc2 = bfc.reshape(1, 256)
    layer_ps = [params['l1'], params['l2'], params['l3'], params['l4']]
    cins = [16, 16, 32, 64]
    couts = [16, 32, 64, 128]
    nr = nb * N

    xt_pad = jnp.pad(jnp.transpose(x, (0, 2, 1)), ((0, 0), (0, 0), (0, 13)))
    aug, xf = _prep(nb, xt_pad, wi_pad, bi2)
    idx = _knn(nb, aug, jnp.transpose(aug, (0, 2, 1)))
    idx_flat = jnp.transpose(idx, (2, 0, 1)).reshape(K * nr)
    posp = aug[:, :, :16].reshape(nr, 16)
    feats = xf.reshape(nr, 16)

    pn3 = None
    os_ = []
    for li in range(4):
        cin, c = cins[li], couts[li]
        if li == 0:
            table = jnp.concatenate([feats, posp], axis=1)
            g = _make_gather(K * nr, 32)(table, idx_flat)
            xn3 = g[:, :16].reshape(K, nr, 16)
            pn3 = g[:, 16:].reshape(K, nr, 16)
        else:
            g = _make_gather(K * nr, cin)(feats, idx_flat)
            xn3 = g.reshape(K, nr, cin)
        feats = _layer(nb, cin, c, feats, xn3, pn3, posp, layer_ps[li])
        os_.append(feats)
    return _pool(nb, os_[0], os_[1], os_[2], os_[3], wfc, bfc2).reshape(nb, 256)


# speedup vs baseline: 7.7857x; 1.1288x over previous
import functools

import jax
import jax.numpy as jnp
from jax import lax
from jax.experimental import pallas as pl
from jax.experimental.pallas import tpu as pltpu
from jax.experimental.pallas import tpu_sc as plsc

N = 4096
K = 16
NW = 32
CH = 128
NFLY = 8
BK = 256
BL = 512
F32 = jnp.float32


def _prep_body(xt_ref, wi_ref, bi_ref, aug_ref, xf_ref):
    xt = xt_ref[0]
    c = jnp.mean(xt, axis=0, keepdims=True)
    xc = xt - c
    sq = jnp.sum(xc * xc, axis=1, keepdims=True)
    md = jnp.max(jnp.sqrt(sq))
    pos = xc / (md + 1e-8)
    psq = jnp.sum(pos * pos, axis=1, keepdims=True)
    aug_ref[0] = jnp.concatenate(
        [pos, psq, jnp.zeros((N, 15), F32)], axis=1)
    xf_ref[0] = (
        jnp.dot(pos, wi_ref[...], preferred_element_type=F32) + bi_ref[...])


def _prep(nb, xt_pad, wi_pad, bi):
    return pl.pallas_call(
        _prep_body,
        grid=(nb,),
        in_specs=[
            pl.BlockSpec((1, N, 16), lambda b: (b, 0, 0)),
            pl.BlockSpec((16, 16), lambda b: (0, 0)),
            pl.BlockSpec((1, 16), lambda b: (0, 0)),
        ],
        out_specs=(
            pl.BlockSpec((1, N, 32), lambda b: (b, 0, 0)),
            pl.BlockSpec((1, N, 16), lambda b: (b, 0, 0)),
        ),
        out_shape=(
            jax.ShapeDtypeStruct((nb, N, 32), F32),
            jax.ShapeDtypeStruct((nb, N, 16), F32),
        ),
    )(xt_pad, wi_pad, bi)


def _knn_body(grow_ref, gt_ref, idx_ref):
    g = grow_ref[0]
    sq = g[:, 16:17]
    a = jnp.concatenate(
        [2.0 * g[:, :16], jnp.full((BK, 1), -1.0, F32),
         jnp.zeros((BK, 15), F32)], axis=1)
    pd = jnp.dot(a, gt_ref[0], preferred_element_type=F32) - sq
    colid = lax.broadcasted_iota(jnp.int32, (BK, N), 1)
    neg = jnp.float32(-jnp.inf)
    outs = []
    for _ in range(K):
        it = jnp.argmax(pd, axis=1, keepdims=True).astype(jnp.int32)
        outs.append(it)
        pd = jnp.where(colid == it, neg, pd)
    idx_ref[0] = jnp.concatenate(outs, axis=1) + pl.program_id(0) * N


def _knn(nb, aug, aug_t):
    return pl.pallas_call(
        _knn_body,
        grid=(nb, N // BK),
        in_specs=[
            pl.BlockSpec((1, BK, 32), lambda b, i: (b, i, 0)),
            pl.BlockSpec((1, 32, N), lambda b, i: (b, 0, 0)),
        ],
        out_specs=pl.BlockSpec((1, BK, K), lambda b, i: (b, i, 0)),
        out_shape=jax.ShapeDtypeStruct((nb, N, K), jnp.int32),
    )(aug, aug_t)


@functools.lru_cache(maxsize=None)
def _make_gather(nr, d):
    half = nr // 2
    sc_rows = CH * NFLY
    nsc = half // sc_rows
    mesh = plsc.VectorSubcoreMesh(core_axis_name="c", subcore_axis_name="s")

    @functools.partial(
        pl.kernel, mesh=mesh,
        out_type=jax.ShapeDtypeStruct((K, nr, d), F32),
        compiler_params=pltpu.CompilerParams(use_tc_tiling_on_sc=False),
        scratch_types=[
            pltpu.VMEM((half,), jnp.int32),
            pltpu.VMEM((sc_rows, d), F32),
            pltpu.SemaphoreType.DMA,
        ],
    )
    def gk(table_hbm, idx_hbm, out_hbm, idx_v, rows_v, sem):
        wid = lax.axis_index("s") * 2 + lax.axis_index("c")
        j = wid // 2
        hbase = (wid % 2) * half
        pltpu.sync_copy(idx_hbm.at[pl.ds(j * nr + hbase, half)], idx_v)

        def body(i, carry):
            off = i * sc_rows
            cps = []
            for q in range(NFLY):
                cps.append(pltpu.async_copy(
                    table_hbm.at[idx_v.at[pl.ds(off + q * CH, CH)]],
                    rows_v.at[pl.ds(q * CH, CH)], sem))
            for cp in cps:
                cp.wait()
            pltpu.sync_copy(rows_v, out_hbm.at[j, pl.ds(hbase + off, sc_rows)])
            return carry

        lax.fori_loop(0, nsc, body, 0)

    return gk


def _layer_body(cin, c, x_ref, xn_ref, pn_ref, pos_ref,
                wc_ref, bc_ref, wn_ref, bn_ref, wp1_ref, bp1_ref,
                wp2_ref, bp2_ref, wa1_ref, ba1_ref, wa2_ref, ba2_ref,
                ws_ref, bs_ref, out_ref):
    b = x_ref.shape[0]
    x = x_ref[...]
    pos = pos_ref[...]
    wc = wc_ref[...]
    wn = wn_ref[...]
    wp1 = wp1_ref[...]
    wp2 = wp2_ref[...]
    wa1 = wa1_ref[...]
    wa2 = wa2_ref[...]
    ws = ws_ref[...]
    bc = bc_ref[...]
    bn = bn_ref[...]
    bp1 = bp1_ref[...]
    bp2 = bp2_ref[...]
    ba1 = ba1_ref[...]
    ba2 = ba2_ref[...]
    bs = bs_ref[...]

    fc = jnp.dot(x, wc, preferred_element_type=F32) + bc
    m = jnp.full((b, 1), -jnp.inf, F32)
    s = jnp.zeros((b, 1), F32)
    agg = jnp.zeros((b, c), F32)
    for j in range(K):
        xnj = xn_ref[j][:, :cin]
        pnj = pn_ref[j][:, 16:32]
        pdj = pos - pnj
        pe = jnp.maximum(
            jnp.dot(pdj, wp1, preferred_element_type=F32) + bp1, 0.0)
        pe = jnp.dot(pe, wp2, preferred_element_type=F32) + bp2
        fn = jnp.dot(xnj, wn, preferred_element_type=F32) + bn
        t = fc - fn + pe
        h = jnp.maximum(
            jnp.dot(t, wa1, preferred_element_type=F32) + ba1, 0.0)
        h = jnp.dot(h, wa2, preferred_element_type=F32) + ba2
        lg = jnp.dot(h, ws, preferred_element_type=F32) + bs
        v = fn + pe
        nm = jnp.maximum(m, lg)
        sc = jnp.exp(m - nm)
        p = jnp.exp(lg - nm)
        s = s * sc + p
        agg = agg * sc + p * v
        m = nm
    out_ref[...] = fc + agg / s


def _layer(nb, cin, c, feats, xn3, pn3, posp, p):
    wc, bc = p['c']
    wn, bn = p['n']
    wp1, bp1 = p['pos1']
    wp2, bp2 = p['pos2']
    wa1, ba1 = p['a1']
    wa2, ba2 = p['a2']
    ws, bs = p['score']
    wp1p = jnp.zeros((16, c), F32).at[:3].set(wp1)
    nr = nb * N
    args = [feats, xn3, pn3, posp,
            wc, bc.reshape(1, c), wn, bn.reshape(1, c),
            wp1p, bp1.reshape(1, c), wp2, bp2.reshape(1, c),
            wa1, ba1.reshape(1, c), wa2, ba2.reshape(1, c),
            ws, bs.reshape(1, 1)]
    w_specs = [pl.BlockSpec(a.shape, lambda i: (0, 0)) for a in args[4:]]
    return pl.pallas_call(
        functools.partial(_layer_body, cin, c),
        grid=(nr // BL,),
        in_specs=[
            pl.BlockSpec((BL, cin), lambda i: (i, 0)),
            pl.BlockSpec((K, BL, xn3.shape[2]), lambda i: (0, i, 0)),
            pl.BlockSpec((K, BL, 32), lambda i: (0, i, 0)),
            pl.BlockSpec((BL, 16), lambda i: (i, 0)),
        ] + w_specs,
        out_specs=pl.BlockSpec((BL, c), lambda i: (i, 0)),
        out_shape=jax.ShapeDtypeStruct((nr, c), F32),
    )(*args)


def _pool_body(o1_ref, o2_ref, o3_ref, o4_ref, wfc_ref, bfc_ref, out_ref):
    cf = jnp.concatenate(
        [o1_ref[...], o2_ref[...], o3_ref[...], o4_ref[...]], axis=1)
    mx = jnp.max(cf, axis=0, keepdims=True)
    mn = jnp.mean(cf, axis=0, keepdims=True)
    g = jnp.concatenate([mx, mn], axis=1)
    out_ref[0] = (
        jnp.dot(g, wfc_ref[...], preferred_element_type=F32) + bfc_ref[...])


def _pool(nb, o1, o2, o3, o4, wfc, bfc):
    return pl.pallas_call(
        _pool_body,
        grid=(nb,),
        in_specs=[
            pl.BlockSpec((N, 16), lambda b: (b, 0)),
            pl.BlockSpec((N, 32), lambda b: (b, 0)),
            pl.BlockSpec((N, 64), lambda b: (b, 0)),
            pl.BlockSpec((N, 128), lambda b: (b, 0)),
            pl.BlockSpec((480, 256), lambda b: (0, 0)),
            pl.BlockSpec((1, 256), lambda b: (0, 0)),
        ],
        out_specs=pl.BlockSpec((1, 1, 256), lambda b: (b, 0, 0)),
        out_shape=jax.ShapeDtypeStruct((nb, 1, 256), F32),
    )(o1, o2, o3, o4, wfc, bfc)


def kernel(x, params):
    nb = x.shape[0]
    wi, bi = params['inp']
    wi_pad = jnp.zeros((16, 16), F32).at[:3].set(wi)
    bi2 = bi.reshape(1, 16)
    wfc, bfc = params['fc']
    bfc2 = bfc.reshape(1, 256)
    layer_ps = [params['l1'], params['l2'], params['l3'], params['l4']]
    cins = [16, 16, 32, 64]
    couts = [16, 32, 64, 128]
    nr = nb * N

    xt_pad = jnp.pad(jnp.transpose(x, (0, 2, 1)), ((0, 0), (0, 0), (0, 13)))
    aug, xf = _prep(nb, xt_pad, wi_pad, bi2)
    idx = _knn(nb, aug, jnp.transpose(aug, (0, 2, 1)))
    idx_flat = jnp.transpose(idx, (2, 0, 1)).reshape(K * nr)
    posp = aug[:, :, :16].reshape(nr, 16)
    feats = xf.reshape(nr, 16)

    pn3 = None
    os_ = []
    for li in range(4):
        cin, c = cins[li], couts[li]
        if li == 0:
            table = jnp.concatenate([feats, posp], axis=1)
            xn3 = _make_gather(nr, 32)(table, idx_flat)
            pn3 = xn3
        else:
            xn3 = _make_gather(nr, cin)(feats, idx_flat)
        feats = _layer(nb, cin, c, feats, xn3, pn3, posp, layer_ps[li])
        os_.append(feats)
    return _pool(nb, os_[0], os_[1], os_[2], os_[3], wfc, bfc2).reshape(nb, 256)
